# Initial kernel scaffold; baseline (speedup 1.0000x reference)
import jax
import jax.numpy as jnp
from jax.experimental import pallas as pl


def kernel(logits, targets):
    raise NotImplementedError("write your pallas kernel here")


# speedup vs baseline: 11.6921x; 11.6921x over previous
import jax
import jax.numpy as jnp
from jax import lax
from jax.experimental import pallas as pl
from jax.experimental.pallas import tpu as pltpu
from jax.experimental.pallas import tpu_sc as plsc

NBINS = 1792
NROWS = 2 * NBINS
LSTRIDE = NROWS + 1
HSIZE = 16 * LSTRIDE
NW = 32
NPIX = 8 * 512 * 512
PER_W = NPIX // NW
CHUNK = 2048
NCHUNK = PER_W // CHUNK


def _prep_body(lref, tref, kref, vref):
    l0 = lref[0, 0]
    l1 = lref[0, 1]
    t = tref[0]
    p = 1.0 / (1.0 + jnp.exp(l0 - l1))
    e = jnp.abs(t.astype(jnp.float32) - p)
    b = jnp.minimum((e * float(NBINS)).astype(jnp.int32), NBINS - 1)
    lane = jax.lax.broadcasted_iota(jnp.int32, (128, 512), 1) & 15
    kref[0] = lane * LSTRIDE + t * NBINS + b
    vref[0] = e


def _prep(logits, t32):
    return pl.pallas_call(
        _prep_body,
        grid=(8, 4),
        in_specs=[
            pl.BlockSpec((1, 2, 128, 512), lambda s, h: (s, 0, h, 0)),
            pl.BlockSpec((1, 128, 512), lambda s, h: (s, h, 0)),
        ],
        out_specs=[
            pl.BlockSpec((1, 128, 512), lambda s, h: (s, h, 0)),
            pl.BlockSpec((1, 128, 512), lambda s, h: (s, h, 0)),
        ],
        out_shape=[
            jax.ShapeDtypeStruct((8, 512, 512), jnp.int32),
            jax.ShapeDtypeStruct((8, 512, 512), jnp.float32),
        ],
    )(logits, t32)


def _hist_body(keys, vals, out, kbuf, vbuf, cnt_v, sum_v):
    wid = lax.axis_index("s") * 2 + lax.axis_index("c")
    base = wid * PER_W
    zeros16 = jnp.zeros((16,), jnp.float32)
    ones16 = jnp.ones((16,), jnp.float32)

    def zbody(i, _):
        cnt_v[pl.ds(i * 16, 16)] = zeros16
        sum_v[pl.ds(i * 16, 16)] = zeros16
        return 0

    lax.fori_loop(0, HSIZE // 16, zbody, 0)

    def cbody(c, _):
        off = pl.multiple_of(base + c * CHUNK, CHUNK)
        pltpu.sync_copy(keys.at[pl.ds(off, CHUNK)], kbuf)
        pltpu.sync_copy(vals.at[pl.ds(off, CHUNK)], vbuf)

        def ibody(i, _):
            kv = kbuf[pl.ds(i * 16, 16)]
            vv = vbuf[pl.ds(i * 16, 16)]
            plsc.addupdate_scatter(sum_v, [kv], vv)
            plsc.addupdate_scatter(cnt_v, [kv], ones16)
            return 0

        lax.fori_loop(0, CHUNK // 16, ibody, 0)
        return 0

    lax.fori_loop(0, NCHUNK, cbody, 0)
    pltpu.sync_copy(cnt_v, out.at[wid, 0])
    pltpu.sync_copy(sum_v, out.at[wid, 1])


def _hist(keys_f, vals_f):
    return pl.kernel(
        _hist_body,
        out_type=jax.ShapeDtypeStruct((NW, 2, HSIZE), jnp.float32),
        mesh=plsc.VectorSubcoreMesh(core_axis_name="c", subcore_axis_name="s"),
        compiler_params=pltpu.CompilerParams(needs_layout_passes=False),
        scratch_types=[
            pltpu.VMEM((CHUNK,), jnp.int32),
            pltpu.VMEM((CHUNK,), jnp.float32),
            pltpu.VMEM((HSIZE,), jnp.float32),
            pltpu.VMEM((HSIZE,), jnp.float32),
        ],
    )(keys_f, vals_f)


def _final_body(href, oref):
    h = href[...]
    col = jax.lax.broadcasted_iota(jnp.int32, (8, 1024), 1)
    srow = jax.lax.broadcasted_iota(jnp.int32, (8, 1024), 0)
    same_s = (col // 128) == srow
    kind = (col // 16) % 2
    sel_cnt = (same_s & (kind == 0)).astype(jnp.float32)
    sel_sum = (same_s & (kind == 1)).astype(jnp.float32)
    dot = lambda a, b: jax.lax.dot_general(
        a, b, (((1,), (0,)), ((), ())),
        preferred_element_type=jnp.float32,
        precision=jax.lax.Precision.HIGHEST)
    cnt = dot(sel_cnt, h)
    ssum = dot(sel_sum, h)
    Q = cnt[:, :NBINS]
    P = cnt[:, NBINS:NROWS]
    SQ = ssum[:, :NBINS]
    SP = ssum[:, NBINS:NROWS]
    G = jnp.sum(P, axis=1, keepdims=True)
    X = jnp.concatenate([Q, P], axis=0)
    u = jax.lax.broadcasted_iota(jnp.int32, (NBINS, NBINS), 0)
    t = jax.lax.broadcasted_iota(jnp.int32, (NBINS, NBINS), 1)
    UT = (u > t).astype(jnp.float32)
    MF = dot(X, UT)
    M = MF[:8]
    F = MF[8:]
    pos_den = jnp.maximum(G + M + 0.5 * Q, 1.0)
    posv = jnp.sum(SP / pos_den, axis=1, keepdims=True)
    d1 = jnp.maximum(G + M, 0.5)
    d2 = jnp.maximum(G + M + Q, 0.5)
    negv = jnp.sum(SQ * (G - F - 0.5 * P) / (d1 * d2), axis=1, keepdims=True)
    lossv = posv + negv
    tb = jax.lax.broadcasted_iota(jnp.int32, (8, NBINS), 1)
    maxb = jnp.max(jnp.where(Q > 0, (tb + 1).astype(jnp.float32), 0.0),
                   axis=1, keepdims=True) / float(NBINS)
    lossv = jnp.where(G > 0.5, lossv, maxb)
    oref[...] = jnp.mean(lossv, keepdims=True)


def _final(h2d):
    return pl.pallas_call(
        _final_body,
        out_shape=jax.ShapeDtypeStruct((1, 1), jnp.float32),
    )(h2d)


def kernel(logits, targets):
    t32 = targets.astype(jnp.int32)
    keys, vals = _prep(logits, t32)
    hist = _hist(keys.reshape(-1), vals.reshape(-1))
    loss = _final(hist.reshape(NW * 2 * 16, LSTRIDE))
    return loss[0, 0]


# speedup vs baseline: 24.3778x; 2.0850x over previous
import jax
import jax.numpy as jnp
from jax import lax
from jax.experimental import pallas as pl
from jax.experimental.pallas import tpu as pltpu
from jax.experimental.pallas import tpu_sc as plsc

NBINS = 1792
NROWS = 2 * NBINS
LSTRIDE = NROWS + 1
HSIZE = 16 * LSTRIDE
NW = 32
NPIX = 8 * 512 * 512
PER_W = NPIX // NW
CHUNK = 2048
NCHUNK = PER_W // CHUNK


def _prep_body(lref, tref, kref):
    l0 = lref[0, 0]
    l1 = lref[0, 1]
    t = tref[0]
    p = 1.0 / (1.0 + jnp.exp(l0 - l1))
    e = jnp.abs(t.astype(jnp.float32) - p)
    b = jnp.minimum((e * float(NBINS)).astype(jnp.int32), NBINS - 1)
    lane = jax.lax.broadcasted_iota(jnp.int32, (128, 512), 1) & 15
    kref[0] = lane * LSTRIDE + t * NBINS + b


def _prep(logits, t32):
    return pl.pallas_call(
        _prep_body,
        grid=(8, 4),
        in_specs=[
            pl.BlockSpec((1, 2, 128, 512), lambda s, h: (s, 0, h, 0)),
            pl.BlockSpec((1, 128, 512), lambda s, h: (s, h, 0)),
        ],
        out_specs=pl.BlockSpec((1, 128, 512), lambda s, h: (s, h, 0)),
        out_shape=jax.ShapeDtypeStruct((8, 512, 512), jnp.int32),
    )(logits, t32)


def _hist_body(keys, out, kb0, kb1, cnt_v, red_v, ks0, ks1):
    wid = lax.axis_index("s") * 2 + lax.axis_index("c")
    base = wid * PER_W
    zeros16 = jnp.zeros((16,), jnp.float32)
    ones16 = jnp.ones((16,), jnp.float32)
    kbufs, ksems = (kb0, kb1), (ks0, ks1)

    def start(c):
        b = c % 2
        off = pl.multiple_of(base + c * CHUNK, CHUNK)
        return pltpu.async_copy(keys.at[pl.ds(off, CHUNK)], kbufs[b], ksems[b])

    pending = start(0)

    def zbody(i, _):
        cnt_v[pl.ds(i * 16, 16)] = zeros16
        return 0

    lax.fori_loop(0, HSIZE // 16, zbody, 0)

    for c in range(NCHUNK):
        pending.wait()
        if c + 1 < NCHUNK:
            nxt = start(c + 1)
        kb = kbufs[c % 2]

        @plsc.parallel_loop(0, CHUNK // 16, 1, unroll=8)
        def _(i, kb=kb):
            kv = kb[pl.ds(i * 16, 16)]
            plsc.addupdate_scatter(cnt_v, [kv], ones16)

        if c + 1 < NCHUNK:
            pending = nxt

    @plsc.parallel_loop(0, NROWS // 16, 1, unroll=2)
    def _(g):
        acc = cnt_v[pl.ds(g * 16, 16)]
        for l in range(1, 16):
            acc = acc + cnt_v[pl.ds(l * LSTRIDE + g * 16, 16)]
        red_v[pl.ds(g * 16, 16)] = acc

    pltpu.sync_copy(red_v, out.at[wid])


def _hist(keys_f):
    return pl.kernel(
        _hist_body,
        out_type=jax.ShapeDtypeStruct((NW, NROWS), jnp.float32),
        mesh=plsc.VectorSubcoreMesh(core_axis_name="c", subcore_axis_name="s"),
        compiler_params=pltpu.CompilerParams(needs_layout_passes=False),
        scratch_types=[
            pltpu.VMEM((CHUNK,), jnp.int32),
            pltpu.VMEM((CHUNK,), jnp.int32),
            pltpu.VMEM((HSIZE,), jnp.float32),
            pltpu.VMEM((NROWS,), jnp.float32),
            pltpu.SemaphoreType.DMA,
            pltpu.SemaphoreType.DMA,
        ],
    )(keys_f)


def _final_body(href, oref):
    h = href[...]
    col = jax.lax.broadcasted_iota(jnp.int32, (8, NW), 1)
    srow = jax.lax.broadcasted_iota(jnp.int32, (8, NW), 0)
    sel = ((col // 4) == srow).astype(jnp.float32)
    dot = lambda a, b: jax.lax.dot_general(
        a, b, (((1,), (0,)), ((), ())),
        preferred_element_type=jnp.float32,
        precision=jax.lax.Precision.HIGHEST)
    cnt = dot(sel, h)
    Q = cnt[:, :NBINS]
    P = cnt[:, NBINS:]
    G = jnp.sum(P, axis=1, keepdims=True)
    ctr = (jax.lax.broadcasted_iota(jnp.int32, (8, NBINS), 1).astype(
        jnp.float32) + 0.5) / float(NBINS)
    SQ = Q * ctr
    SP = P * ctr
    X = jnp.concatenate([Q, P], axis=0)
    u = jax.lax.broadcasted_iota(jnp.int32, (NBINS, NBINS), 0)
    t = jax.lax.broadcasted_iota(jnp.int32, (NBINS, NBINS), 1)
    UT = (u > t).astype(jnp.float32)
    MF = dot(X, UT)
    M = MF[:8]
    F = MF[8:]
    pos_den = jnp.maximum(G + M + 0.5 * Q, 1.0)
    posv = jnp.sum(SP / pos_den, axis=1, keepdims=True)
    d1 = jnp.maximum(G + M, 0.5)
    d2 = jnp.maximum(G + M + Q, 0.5)
    negv = jnp.sum(SQ * (G - F - 0.5 * P) / (d1 * d2), axis=1, keepdims=True)
    lossv = posv + negv
    tb = jax.lax.broadcasted_iota(jnp.int32, (8, NBINS), 1)
    maxb = jnp.max(jnp.where(Q > 0, (tb + 1).astype(jnp.float32), 0.0),
                   axis=1, keepdims=True) / float(NBINS)
    lossv = jnp.where(G > 0.5, lossv, maxb)
    oref[...] = jnp.mean(lossv, keepdims=True)


def _final(hw):
    return pl.pallas_call(
        _final_body,
        out_shape=jax.ShapeDtypeStruct((1, 1), jnp.float32),
    )(hw)


def kernel(logits, targets):
    t32 = targets.astype(jnp.int32)
    keys = _prep(logits, t32)
    hist = _hist(keys.reshape(-1))
    loss = _final(hist)
    return loss[0, 0]


# speedup vs baseline: 29.0104x; 1.1900x over previous
import jax
import jax.numpy as jnp
from jax import lax
from jax.experimental import pallas as pl
from jax.experimental.pallas import tpu as pltpu
from jax.experimental.pallas import tpu_sc as plsc

NBINS = 1792
NROWS = 2 * NBINS
LSTRIDE = NROWS + 1
HSIZE = 16 * LSTRIDE
NW = 16
NPIX = 8 * 512 * 512
PER_W = NPIX // NW
CHUNK = 2048
WORDS_W = PER_W // 2
NCHUNK = WORDS_W // CHUNK
NRING = 4


def _prep_body(lref, tref, kref):
    l0 = lref[0, 0]
    l1 = lref[0, 1]
    t = tref[0]
    p = 1.0 / (1.0 + jnp.exp(l0 - l1))
    e = jnp.abs(t.astype(jnp.float32) - p)
    b = jnp.minimum((e * float(NBINS)).astype(jnp.int32), NBINS - 1)
    lane = jax.lax.broadcasted_iota(jnp.int32, (128, 512), 1) & 15
    addr = lane * LSTRIDE + t * NBINS + b
    kref[0] = addr[:, :256] | (addr[:, 256:] << 16)


def _prep(logits, t32):
    return pl.pallas_call(
        _prep_body,
        grid=(8, 4),
        in_specs=[
            pl.BlockSpec((1, 2, 128, 512), lambda s, h: (s, 0, h, 0)),
            pl.BlockSpec((1, 128, 512), lambda s, h: (s, h, 0)),
        ],
        out_specs=pl.BlockSpec((1, 128, 256), lambda s, h: (s, h, 0)),
        out_shape=jax.ShapeDtypeStruct((8, 512, 256), jnp.int32),
    )(logits, t32)


def _hist_body(keys, out, kb0, kb1, kb2, kb3, cnt_v, red_v,
               ks0, ks1, ks2, ks3):
    wid = lax.axis_index("s")
    base = wid * WORDS_W
    zeros16 = jnp.zeros((16,), jnp.float32)
    ones16 = jnp.ones((16,), jnp.float32)
    kbufs, ksems = (kb0, kb1, kb2, kb3), (ks0, ks1, ks2, ks3)

    def start(c):
        b = c % NRING
        off = pl.multiple_of(base + c * CHUNK, CHUNK)
        return pltpu.async_copy(keys.at[pl.ds(off, CHUNK)], kbufs[b], ksems[b])

    handles = {c: start(c) for c in range(min(NRING - 1, NCHUNK))}

    def zbody(i, _):
        cnt_v[pl.ds(i * 16, 16)] = zeros16
        return 0

    lax.fori_loop(0, HSIZE // 16, zbody, 0)

    for c in range(NCHUNK):
        handles.pop(c).wait()
        if c + NRING - 1 < NCHUNK:
            handles[c + NRING - 1] = start(c + NRING - 1)
        kb = kbufs[c % NRING]

        @plsc.parallel_loop(0, CHUNK // 16, 1, unroll=8)
        def _(i, kb=kb):
            w = kb[pl.ds(i * 16, 16)]
            klo = w & 0xFFFF
            khi = lax.shift_right_logical(w, 16)
            plsc.addupdate_scatter(cnt_v, [klo], ones16)
            plsc.addupdate_scatter(cnt_v, [khi], ones16)

    @plsc.parallel_loop(0, NROWS // 16, 1, unroll=2)
    def _(g):
        acc = cnt_v[pl.ds(g * 16, 16)]
        for l in range(1, 16):
            acc = acc + cnt_v[pl.ds(l * LSTRIDE + g * 16, 16)]
        red_v[pl.ds(g * 16, 16)] = acc

    pltpu.sync_copy(red_v, out.at[wid])


def _hist(keys_f):
    return pl.kernel(
        _hist_body,
        out_type=jax.ShapeDtypeStruct((NW, NROWS), jnp.float32),
        mesh=plsc.VectorSubcoreMesh(core_axis_name="c", subcore_axis_name="s",
                                    num_cores=1),
        compiler_params=pltpu.CompilerParams(needs_layout_passes=False),
        scratch_types=[
            pltpu.VMEM((CHUNK,), jnp.int32),
            pltpu.VMEM((CHUNK,), jnp.int32),
            pltpu.VMEM((CHUNK,), jnp.int32),
            pltpu.VMEM((CHUNK,), jnp.int32),
            pltpu.VMEM((HSIZE,), jnp.float32),
            pltpu.VMEM((NROWS,), jnp.float32),
            pltpu.SemaphoreType.DMA,
            pltpu.SemaphoreType.DMA,
            pltpu.SemaphoreType.DMA,
            pltpu.SemaphoreType.DMA,
        ],
    )(keys_f)


def _final_body(href, oref):
    h = href[...]
    col = jax.lax.broadcasted_iota(jnp.int32, (8, NW), 1)
    srow = jax.lax.broadcasted_iota(jnp.int32, (8, NW), 0)
    sel = ((col // (NW // 8)) == srow).astype(jnp.float32)
    dot = lambda a, b: jax.lax.dot_general(
        a, b, (((1,), (0,)), ((), ())),
        preferred_element_type=jnp.float32,
        precision=jax.lax.Precision.HIGHEST)
    cnt = dot(sel, h)
    Q = cnt[:, :NBINS]
    P = cnt[:, NBINS:]
    G = jnp.sum(P, axis=1, keepdims=True)
    ctr = (jax.lax.broadcasted_iota(jnp.int32, (8, NBINS), 1).astype(
        jnp.float32) + 0.5) / float(NBINS)
    SQ = Q * ctr
    SP = P * ctr
    X = jnp.concatenate([Q, P], axis=0)
    u = jax.lax.broadcasted_iota(jnp.int32, (NBINS, NBINS), 0)
    t = jax.lax.broadcasted_iota(jnp.int32, (NBINS, NBINS), 1)
    UT = (u > t).astype(jnp.float32)
    MF = dot(X, UT)
    M = MF[:8]
    F = MF[8:]
    pos_den = jnp.maximum(G + M + 0.5 * Q, 1.0)
    posv = jnp.sum(SP / pos_den, axis=1, keepdims=True)
    d1 = jnp.maximum(G + M, 0.5)
    d2 = jnp.maximum(G + M + Q, 0.5)
    negv = jnp.sum(SQ * (G - F - 0.5 * P) / (d1 * d2), axis=1, keepdims=True)
    lossv = posv + negv
    tb = jax.lax.broadcasted_iota(jnp.int32, (8, NBINS), 1)
    maxb = jnp.max(jnp.where(Q > 0, (tb + 1).astype(jnp.float32), 0.0),
                   axis=1, keepdims=True) / float(NBINS)
    lossv = jnp.where(G > 0.5, lossv, maxb)
    oref[...] = jnp.mean(lossv, keepdims=True)


def _final(hw):
    return pl.pallas_call(
        _final_body,
        out_shape=jax.ShapeDtypeStruct((1, 1), jnp.float32),
    )(hw)


def kernel(logits, targets):
    t32 = targets.astype(jnp.int32)
    packed = _prep(logits, t32)
    hist = _hist(packed.reshape(-1))
    loss = _final(hist)
    return loss[0, 0]


# speedup vs baseline: 31.2753x; 1.0781x over previous
import jax
import jax.numpy as jnp
from jax import lax
from jax.experimental import pallas as pl
from jax.experimental.pallas import tpu as pltpu
from jax.experimental.pallas import tpu_sc as plsc

NBINS = 1792
NROWS = 2 * NBINS
LSTRIDE = NROWS + 1
HSIZE = 16 * LSTRIDE
NW = 32
NPIX = 8 * 512 * 512
PER_W = NPIX // NW
CHUNK = 2048
WORDS_W = PER_W // 2
NCHUNK = WORDS_W // CHUNK
NRING = 6


def _prep_body(lref, tref, kref):
    l0 = lref[0, 0]
    l1 = lref[0, 1]
    t = tref[0]
    p = 1.0 / (1.0 + jnp.exp(l0 - l1))
    e = jnp.abs(t.astype(jnp.float32) - p)
    b = jnp.minimum((e * float(NBINS)).astype(jnp.int32), NBINS - 1)
    lane = jax.lax.broadcasted_iota(jnp.int32, (128, 512), 1) & 15
    addr = lane * LSTRIDE + t * NBINS + b
    kref[0] = addr[:, :256] | (addr[:, 256:] << 16)


def _prep(logits, t32):
    return pl.pallas_call(
        _prep_body,
        grid=(8, 4),
        in_specs=[
            pl.BlockSpec((1, 2, 128, 512), lambda s, h: (s, 0, h, 0)),
            pl.BlockSpec((1, 128, 512), lambda s, h: (s, h, 0)),
        ],
        out_specs=pl.BlockSpec((1, 128, 256), lambda s, h: (s, h, 0)),
        out_shape=jax.ShapeDtypeStruct((8, 512, 256), jnp.int32),
    )(logits, t32)


def _hist_body(keys, out, kb0, kb1, kb2, kb3, kb4, kb5, cnt_v, red_v,
               ks0, ks1, ks2, ks3, ks4, ks5):
    wid = lax.axis_index("s") * 2 + lax.axis_index("c")
    base = wid * WORDS_W
    zeros16 = jnp.zeros((16,), jnp.float32)
    ones16 = jnp.ones((16,), jnp.float32)
    kbufs = (kb0, kb1, kb2, kb3, kb4, kb5)
    ksems = (ks0, ks1, ks2, ks3, ks4, ks5)

    def start(c):
        b = c % NRING
        off = pl.multiple_of(base + c * CHUNK, CHUNK)
        return pltpu.async_copy(keys.at[pl.ds(off, CHUNK)], kbufs[b], ksems[b])

    handles = {c: start(c) for c in range(min(NRING - 1, NCHUNK))}

    def zbody(i, _):
        cnt_v[pl.ds(i * 16, 16)] = zeros16
        return 0

    lax.fori_loop(0, HSIZE // 16, zbody, 0)

    for c in range(NCHUNK):
        handles.pop(c).wait()
        if c + NRING - 1 < NCHUNK:
            handles[c + NRING - 1] = start(c + NRING - 1)
        kb = kbufs[c % NRING]

        @plsc.parallel_loop(0, CHUNK // 16, 1, unroll=8)
        def _(i, kb=kb):
            w = kb[pl.ds(i * 16, 16)]
            klo = w & 0xFFFF
            khi = lax.shift_right_logical(w, 16)
            plsc.addupdate_scatter(cnt_v, [klo], ones16)
            plsc.addupdate_scatter(cnt_v, [khi], ones16)

    @plsc.parallel_loop(0, NROWS // 16, 1, unroll=2)
    def _(g):
        acc = cnt_v[pl.ds(g * 16, 16)]
        for l in range(1, 16):
            acc = acc + cnt_v[pl.ds(l * LSTRIDE + g * 16, 16)]
        red_v[pl.ds(g * 16, 16)] = acc

    pltpu.sync_copy(red_v, out.at[wid])


def _hist(keys_f):
    return pl.kernel(
        _hist_body,
        out_type=jax.ShapeDtypeStruct((NW, NROWS), jnp.float32),
        mesh=plsc.VectorSubcoreMesh(core_axis_name="c", subcore_axis_name="s"),
        compiler_params=pltpu.CompilerParams(needs_layout_passes=False),
        scratch_types=[
            pltpu.VMEM((CHUNK,), jnp.int32),
            pltpu.VMEM((CHUNK,), jnp.int32),
            pltpu.VMEM((CHUNK,), jnp.int32),
            pltpu.VMEM((CHUNK,), jnp.int32),
            pltpu.VMEM((CHUNK,), jnp.int32),
            pltpu.VMEM((CHUNK,), jnp.int32),
            pltpu.VMEM((HSIZE,), jnp.float32),
            pltpu.VMEM((NROWS,), jnp.float32),
            pltpu.SemaphoreType.DMA,
            pltpu.SemaphoreType.DMA,
            pltpu.SemaphoreType.DMA,
            pltpu.SemaphoreType.DMA,
            pltpu.SemaphoreType.DMA,
            pltpu.SemaphoreType.DMA,
        ],
    )(keys_f)


def _final_body(href, oref):
    h = href[...]
    col = jax.lax.broadcasted_iota(jnp.int32, (8, NW), 1)
    srow = jax.lax.broadcasted_iota(jnp.int32, (8, NW), 0)
    sel = ((col // 4) == srow).astype(jnp.float32)
    dot = lambda a, b: jax.lax.dot_general(
        a, b, (((1,), (0,)), ((), ())),
        preferred_element_type=jnp.float32,
        precision=jax.lax.Precision.HIGHEST)
    cnt = dot(sel, h)
    Q = cnt[:, :NBINS]
    P = cnt[:, NBINS:]
    G = jnp.sum(P, axis=1, keepdims=True)
    ctr = (jax.lax.broadcasted_iota(jnp.int32, (8, NBINS), 1).astype(
        jnp.float32) + 0.5) / float(NBINS)
    SQ = Q * ctr
    SP = P * ctr
    X = jnp.concatenate([Q, P], axis=0)
    u = jax.lax.broadcasted_iota(jnp.int32, (NBINS, NBINS), 0)
    t = jax.lax.broadcasted_iota(jnp.int32, (NBINS, NBINS), 1)
    UT = (u > t).astype(jnp.float32)
    MF = dot(X, UT)
    M = MF[:8]
    F = MF[8:]
    pos_den = jnp.maximum(G + M + 0.5 * Q, 1.0)
    posv = jnp.sum(SP / pos_den, axis=1, keepdims=True)
    d1 = jnp.maximum(G + M, 0.5)
    d2 = jnp.maximum(G + M + Q, 0.5)
    negv = jnp.sum(SQ * (G - F - 0.5 * P) / (d1 * d2), axis=1, keepdims=True)
    lossv = posv + negv
    tb = jax.lax.broadcasted_iota(jnp.int32, (8, NBINS), 1)
    maxb = jnp.max(jnp.where(Q > 0, (tb + 1).astype(jnp.float32), 0.0),
                   axis=1, keepdims=True) / float(NBINS)
    lossv = jnp.where(G > 0.5, lossv, maxb)
    oref[...] = jnp.mean(lossv, keepdims=True)


def _final(hw):
    return pl.pallas_call(
        _final_body,
        out_shape=jax.ShapeDtypeStruct((1, 1), jnp.float32),
    )(hw)


def kernel(logits, targets):
    t32 = targets.astype(jnp.int32)
    packed = _prep(logits, t32)
    hist = _hist(packed.reshape(-1))
    loss = _final(hist)
    return loss[0, 0]


# speedup vs baseline: 31.2863x; 1.0004x over previous
import jax
import jax.numpy as jnp
from jax import lax
from jax.experimental import pallas as pl
from jax.experimental.pallas import tpu as pltpu
from jax.experimental.pallas import tpu_sc as plsc

NBINS = 1792
NROWS = 2 * NBINS
LSTRIDE = NROWS + 1
HSIZE = 16 * LSTRIDE
NW = 32
NPIX = 8 * 512 * 512
PER_W = NPIX // NW
CHUNK = 2048
WORDS_W = PER_W // 2
NCHUNK = WORDS_W // CHUNK
NRING = 4


def _prep_body(lref, tref, kref):
    l0 = lref[0, 0]
    l1 = lref[0, 1]
    t = tref[0]
    p = 1.0 / (1.0 + jnp.exp(l0 - l1))
    e = jnp.abs(t.astype(jnp.float32) - p)
    b = jnp.minimum((e * float(NBINS)).astype(jnp.int32), NBINS - 1)
    lane = jax.lax.broadcasted_iota(jnp.int32, (128, 512), 1) & 15
    addr = lane * LSTRIDE + t * NBINS + b
    kref[0] = addr[:, :256] | (addr[:, 256:] << 16)


def _prep(logits, t32):
    return pl.pallas_call(
        _prep_body,
        grid=(8, 4),
        in_specs=[
            pl.BlockSpec((1, 2, 128, 512), lambda s, h: (s, 0, h, 0)),
            pl.BlockSpec((1, 128, 512), lambda s, h: (s, h, 0)),
        ],
        out_specs=pl.BlockSpec((1, 128, 256), lambda s, h: (s, h, 0)),
        out_shape=jax.ShapeDtypeStruct((8, 512, 256), jnp.int32),
    )(logits, t32)


def _hist_body(keys, out, kb0, kb1, kb2, kb3, cnt_v, red_v,
               ks0, ks1, ks2, ks3):
    wid = lax.axis_index("s") * 2 + lax.axis_index("c")
    base = wid * WORDS_W
    zeros16 = jnp.zeros((16,), jnp.float32)
    ones16 = jnp.ones((16,), jnp.float32)
    kbufs, ksems = (kb0, kb1, kb2, kb3), (ks0, ks1, ks2, ks3)

    def start(c):
        b = c % NRING
        off = pl.multiple_of(base + c * CHUNK, CHUNK)
        return pltpu.async_copy(keys.at[pl.ds(off, CHUNK)], kbufs[b], ksems[b])

    handles = {c: start(c) for c in range(min(NRING - 1, NCHUNK))}

    def zbody(i, _):
        cnt_v[pl.ds(i * 16, 16)] = zeros16
        return 0

    lax.fori_loop(0, HSIZE // 16, zbody, 0)

    for c in range(NCHUNK):
        handles.pop(c).wait()
        if c + NRING - 1 < NCHUNK:
            handles[c + NRING - 1] = start(c + NRING - 1)
        kb = kbufs[c % NRING]

        @plsc.parallel_loop(0, CHUNK // 16, 1, unroll=8)
        def _(i, kb=kb):
            w = kb[pl.ds(i * 16, 16)]
            klo = w & 0xFFFF
            khi = lax.shift_right_logical(w, 16)
            plsc.addupdate_scatter(cnt_v, [klo], ones16)
            plsc.addupdate_scatter(cnt_v, [khi], ones16)

    @plsc.parallel_loop(0, NROWS // 16, 1, unroll=2)
    def _(g):
        acc = cnt_v[pl.ds(g * 16, 16)]
        for l in range(1, 16):
            acc = acc + cnt_v[pl.ds(l * LSTRIDE + g * 16, 16)]
        red_v[pl.ds(g * 16, 16)] = acc

    pltpu.sync_copy(red_v, out.at[wid])


def _hist(keys_f):
    return pl.kernel(
        _hist_body,
        out_type=jax.ShapeDtypeStruct((NW, NROWS), jnp.float32),
        mesh=plsc.VectorSubcoreMesh(core_axis_name="c", subcore_axis_name="s"),
        compiler_params=pltpu.CompilerParams(needs_layout_passes=False),
        scratch_types=[
            pltpu.VMEM((CHUNK,), jnp.int32),
            pltpu.VMEM((CHUNK,), jnp.int32),
            pltpu.VMEM((CHUNK,), jnp.int32),
            pltpu.VMEM((CHUNK,), jnp.int32),
            pltpu.VMEM((HSIZE,), jnp.float32),
            pltpu.VMEM((NROWS,), jnp.float32),
            pltpu.SemaphoreType.DMA,
            pltpu.SemaphoreType.DMA,
            pltpu.SemaphoreType.DMA,
            pltpu.SemaphoreType.DMA,
        ],
    )(keys_f)


def _final_body(href, oref):
    h = href[...]
    col = jax.lax.broadcasted_iota(jnp.int32, (8, NW), 1)
    srow = jax.lax.broadcasted_iota(jnp.int32, (8, NW), 0)
    sel = ((col // 4) == srow).astype(jnp.float32)
    dot = lambda a, b: jax.lax.dot_general(
        a, b, (((1,), (0,)), ((), ())),
        preferred_element_type=jnp.float32,
        precision=jax.lax.Precision.HIGHEST)
    cnt = dot(sel, h)
    Q = cnt[:, :NBINS]
    P = cnt[:, NBINS:]
    G = jnp.sum(P, axis=1, keepdims=True)
    ctr = (jax.lax.broadcasted_iota(jnp.int32, (8, NBINS), 1).astype(
        jnp.float32) + 0.5) / float(NBINS)
    SQ = Q * ctr
    SP = P * ctr
    X = jnp.concatenate([Q, P], axis=0)
    u = jax.lax.broadcasted_iota(jnp.int32, (NBINS, NBINS), 0)
    t = jax.lax.broadcasted_iota(jnp.int32, (NBINS, NBINS), 1)
    UT = (u > t).astype(jnp.float32)
    MF = dot(X, UT)
    M = MF[:8]
    F = MF[8:]
    pos_den = jnp.maximum(G + M + 0.5 * Q, 1.0)
    posv = jnp.sum(SP / pos_den, axis=1, keepdims=True)
    d1 = jnp.maximum(G + M, 0.5)
    d2 = jnp.maximum(G + M + Q, 0.5)
    negv = jnp.sum(SQ * (G - F - 0.5 * P) / (d1 * d2), axis=1, keepdims=True)
    lossv = posv + negv
    tb = jax.lax.broadcasted_iota(jnp.int32, (8, NBINS), 1)
    maxb = jnp.max(jnp.where(Q > 0, (tb + 1).astype(jnp.float32), 0.0),
                   axis=1, keepdims=True) / float(NBINS)
    lossv = jnp.where(G > 0.5, lossv, maxb)
    oref[...] = jnp.mean(lossv, keepdims=True)


def _final(hw):
    return pl.pallas_call(
        _final_body,
        out_shape=jax.ShapeDtypeStruct((1, 1), jnp.float32),
    )(hw)


def kernel(logits, targets):
    t32 = targets.astype(jnp.int32)
    packed = _prep(logits, t32)
    hist = _hist(packed.reshape(-1))
    loss = _final(hist)
    return loss[0, 0]
